# baseline (speedup 1.0000x reference)
import functools

import jax
import jax.numpy as jnp
from jax import lax
from jax.experimental import pallas as pl
from jax.experimental.pallas import tpu as pltpu
from jax.experimental.pallas import tpu_sc as plsc

N_NODES = 100000
D = 572
N_REL = 6
E = 12000
E3 = 3 * E
E2 = 2 * E
DP = 640

R_CHUNK = 2
N_CHUNKS = N_REL // R_CHUNK
ROWS_C = R_CHUNK * (E3 + E2)
SRC_C = R_CHUNK * E3

NW = 32
CHUNK = 88
PER_W = 3784
ITERS = PER_W // CHUNK
N_PAD_C = NW * PER_W


def _sc_gather(table, idx):
  mesh = plsc.VectorSubcoreMesh(core_axis_name="c", subcore_axis_name="s")

  @functools.partial(
      pl.kernel,
      out_type=jax.ShapeDtypeStruct((N_PAD_C, DP), jnp.float32),
      mesh=mesh,
      scratch_types=[
          pltpu.VMEM((ITERS, CHUNK), jnp.int32),
          pltpu.VMEM((CHUNK, DP), jnp.float32),
          pltpu.VMEM((CHUNK, DP), jnp.float32),
          pltpu.SemaphoreType.DMA,
          pltpu.SemaphoreType.DMA,
      ],
  )
  def k(table_hbm, idx_hbm, out_hbm, idx_v, rows0, rows1, sem0, sem1):
    cid = lax.axis_index("c")
    sid = lax.axis_index("s")
    wid = sid * 2 + cid
    base = wid * PER_W
    pltpu.sync_copy(idx_hbm.at[wid], idx_v)

    def gather(it, buf, sem):
      return pltpu.async_copy(table_hbm.at[idx_v.at[it]], buf, sem)

    def put(it, buf):
      pltpu.sync_copy(buf, out_hbm.at[pl.ds(base + it * CHUNK, CHUNK)])

    gather(0, rows0, sem0)

    def body(hk, carry):
      it0 = 2 * hk
      pltpu.make_async_copy(table_hbm.at[idx_v.at[it0]], rows0, sem0).wait()

      @pl.when(it0 + 1 < ITERS)
      def _():
        gather(it0 + 1, rows1, sem1)

      put(it0, rows0)

      @pl.when(it0 + 1 < ITERS)
      def _():
        pltpu.make_async_copy(
            table_hbm.at[idx_v.at[it0 + 1]], rows1, sem1).wait()

        @pl.when(it0 + 2 < ITERS)
        def _():
          gather(it0 + 2, rows0, sem0)

        put(it0 + 1, rows1)

      return carry

    lax.fori_loop(0, (ITERS + 1) // 2, body, 0, unroll=False)

  return k(table, idx)


BE = 2000
NB = E3 // BE
NB_DST = E2 // BE


def _score_body(src_ref, dst_ref, fake_ref, rel_ref, out_ref):
  j = pl.program_id(1)
  s = jnp.dot(src_ref[...], rel_ref[0], preferred_element_type=jnp.float32)

  @pl.when(j < NB_DST)
  def _():
    out_ref[...] = jnp.sum(s * dst_ref[...], axis=1).reshape(1, 1, BE)

  @pl.when(j >= NB_DST)
  def _():
    out_ref[...] = jnp.sum(s[:, :D] * fake_ref[0], axis=1).reshape(1, 1, BE)


def _score_tc(gathered_c, fake, rel_mat_p, c):
  out = pl.pallas_call(
      _score_body,
      grid=(R_CHUNK, NB),
      in_specs=[
          pl.BlockSpec((BE, DP), lambda i, j: (i * NB + j, 0)),
          pl.BlockSpec(
              (BE, DP),
              lambda i, j: (SRC_C // BE + i * NB_DST
                            + jnp.minimum(j, NB_DST - 1), 0)),
          pl.BlockSpec(
              (1, BE, D),
              lambda i, j: (c * R_CHUNK + i, jnp.maximum(j - NB_DST, 0), 0)),
          pl.BlockSpec((1, DP, DP), lambda i, j: (c * R_CHUNK + i, 0, 0)),
      ],
      out_specs=pl.BlockSpec((1, 1, BE), lambda i, j: (i * NB + j, 0, 0)),
      out_shape=jax.ShapeDtypeStruct((R_CHUNK * NB, 1, BE), jnp.float32),
  )(gathered_c, gathered_c, fake, rel_mat_p)
  return out.reshape(R_CHUNK, E3)


BN = 2000
NBN = N_NODES // BN


def _graph_body(ne_ref, w1_ref, b1_ref, w2_ref, b2_ref, out_ref, acc_ref):
  k = pl.program_id(0)

  @pl.when(k == 0)
  def _():
    acc_ref[...] = jnp.zeros_like(acc_ref)

  acc_ref[...] += jnp.sum(ne_ref[...], axis=0, keepdims=True)

  @pl.when(k == NBN - 1)
  def _():
    hg = acc_ref[:, :D] * jnp.float32(1.0 / N_NODES)
    hgb = hg.astype(jnp.bfloat16).astype(jnp.float32)
    w1b = w1_ref[...].astype(jnp.bfloat16).astype(jnp.float32)
    h1 = jnp.maximum(
        jnp.dot(hgb, w1b, preferred_element_type=jnp.float32,
                precision=lax.Precision.HIGHEST)
        + b1_ref[...], 0.0)
    out_ref[...] = (
        jnp.dot(h1, w2_ref[...], preferred_element_type=jnp.float32,
                precision=lax.Precision.HIGHEST)
        + b2_ref[...])


def _graph_tc(node_emb_p, w1, b1, w2, b2):
  out = pl.pallas_call(
      _graph_body,
      grid=(NBN,),
      in_specs=[
          pl.BlockSpec((BN, DP), lambda k: (k, 0)),
          pl.BlockSpec((D, D // 2), lambda k: (0, 0)),
          pl.BlockSpec((1, D // 2), lambda k: (0, 0)),
          pl.BlockSpec((D // 2, 1), lambda k: (0, 0)),
          pl.BlockSpec((1, 1), lambda k: (0, 0)),
      ],
      out_specs=pl.BlockSpec((1, 1), lambda k: (0, 0)),
      out_shape=jax.ShapeDtypeStruct((1, 1), jnp.float32),
      scratch_shapes=[pltpu.VMEM((1, DP), jnp.float32)],
  )(node_emb_p, w1, b1.reshape(1, -1), w2, b2.reshape(1, 1))
  return out.reshape(1)


def kernel(node_emb, rel_mat, W1, b1, W2, b2, generate_neighbor_emb,
           pos_edges, neg1_edges, neg2_edges):
  src_idx = jnp.concatenate(
      [pos_edges[:, 0], neg1_edges[:, 0], neg2_edges[:, 0]], axis=1)
  dst_idx = jnp.concatenate([pos_edges[:, 1], neg1_edges[:, 1]], axis=1)

  rel_mat_p = jnp.pad(rel_mat, ((0, 0), (0, DP - D), (0, DP - D)))
  node_emb_p = jnp.pad(node_emb, ((0, 0), (0, DP - D)))
  graph_embd = _graph_tc(node_emb_p, W1, b1, W2, b2)

  chunk_scores = []
  for c in range(N_CHUNKS):
    idx_c = jnp.concatenate([
        src_idx[c * R_CHUNK:(c + 1) * R_CHUNK].reshape(-1),
        dst_idx[c * R_CHUNK:(c + 1) * R_CHUNK].reshape(-1),
    ])
    idx_c = jnp.pad(idx_c, (0, N_PAD_C - ROWS_C)).reshape(NW, ITERS, CHUNK)
    gathered_c = _sc_gather(node_emb_p, idx_c)
    chunk_scores.append(
        _score_tc(gathered_c, generate_neighbor_emb, rel_mat_p, c))

  scores = jnp.concatenate(chunk_scores, axis=0)
  pos_score = scores[:, :E].reshape(-1)
  neg_score1 = scores[:, E:2 * E].reshape(-1)
  neg_score2 = scores[:, 2 * E:].reshape(-1)
  return (pos_score, neg_score1, neg_score2, graph_embd)


# speedup vs baseline: 1.5733x; 1.5733x over previous
import functools

import jax
import jax.numpy as jnp
from jax import lax
from jax.experimental import pallas as pl
from jax.experimental.pallas import tpu as pltpu
from jax.experimental.pallas import tpu_sc as plsc

N_NODES = 100000
D = 572
N_REL = 6
E = 12000
E3 = 3 * E
E2 = 2 * E
DP = 640

R_CHUNK = 2
N_CHUNKS = N_REL // R_CHUNK
ROWS_C = R_CHUNK * (E3 + E2)
SRC_C = R_CHUNK * E3

NW = 32
CHUNK = 88
PER_W = 3784
ITERS = PER_W // CHUNK
N_PAD_C = NW * PER_W


def _sc_gather(table, idx):
  mesh = plsc.VectorSubcoreMesh(core_axis_name="c", subcore_axis_name="s")

  @functools.partial(
      pl.kernel,
      out_type=jax.ShapeDtypeStruct((N_PAD_C, DP), jnp.float32),
      mesh=mesh,
      scratch_types=[
          pltpu.VMEM((ITERS, CHUNK), jnp.int32),
          pltpu.VMEM((CHUNK, DP), jnp.float32),
          pltpu.VMEM((CHUNK, DP), jnp.float32),
          pltpu.SemaphoreType.DMA,
          pltpu.SemaphoreType.DMA,
      ],
  )
  def k(table_hbm, idx_hbm, out_hbm, idx_v, rows0, rows1, sem0, sem1):
    cid = lax.axis_index("c")
    sid = lax.axis_index("s")
    wid = sid * 2 + cid
    base = wid * PER_W
    pltpu.sync_copy(idx_hbm.at[wid], idx_v)

    def gather(it, buf, sem):
      return pltpu.async_copy(table_hbm.at[idx_v.at[it]], buf, sem)

    def put(it, buf):
      pltpu.sync_copy(buf, out_hbm.at[pl.ds(base + it * CHUNK, CHUNK)])

    gather(0, rows0, sem0)

    def body(hk, carry):
      it0 = 2 * hk
      pltpu.make_async_copy(table_hbm.at[idx_v.at[it0]], rows0, sem0).wait()

      @pl.when(it0 + 1 < ITERS)
      def _():
        gather(it0 + 1, rows1, sem1)

      put(it0, rows0)

      @pl.when(it0 + 1 < ITERS)
      def _():
        pltpu.make_async_copy(
            table_hbm.at[idx_v.at[it0 + 1]], rows1, sem1).wait()

        @pl.when(it0 + 2 < ITERS)
        def _():
          gather(it0 + 2, rows0, sem0)

        put(it0 + 1, rows1)

      return carry

    lax.fori_loop(0, (ITERS + 1) // 2, body, 0, unroll=False)

  return k(table, idx)


BE = 2000
NB = E3 // BE
NB_DST = E2 // BE


def _score_body(src_ref, dst_ref, fake_ref, rel_ref, out_ref):
  j = pl.program_id(1)
  s = jnp.dot(src_ref[...], rel_ref[0], preferred_element_type=jnp.float32)

  @pl.when(j < NB_DST)
  def _():
    out_ref[...] = jnp.sum(s * dst_ref[...], axis=1).reshape(1, 1, BE)

  @pl.when(j >= NB_DST)
  def _():
    out_ref[...] = jnp.sum(s[:, :D] * fake_ref[0], axis=1).reshape(1, 1, BE)


def _score_tc(gathered_c, fake, rel_mat_p, c):
  out = pl.pallas_call(
      _score_body,
      grid=(R_CHUNK, NB),
      in_specs=[
          pl.BlockSpec((BE, DP), lambda i, j: (i * NB + j, 0)),
          pl.BlockSpec(
              (BE, DP),
              lambda i, j: (SRC_C // BE + i * NB_DST
                            + jnp.minimum(j, NB_DST - 1), 0)),
          pl.BlockSpec(
              (1, BE, D),
              lambda i, j: (c * R_CHUNK + i, jnp.maximum(j - NB_DST, 0), 0)),
          pl.BlockSpec((1, DP, DP), lambda i, j: (c * R_CHUNK + i, 0, 0)),
      ],
      out_specs=pl.BlockSpec((1, 1, BE), lambda i, j: (i * NB + j, 0, 0)),
      out_shape=jax.ShapeDtypeStruct((R_CHUNK * NB, 1, BE), jnp.float32),
  )(gathered_c, gathered_c, fake, rel_mat_p)
  return out.reshape(R_CHUNK, E3)


BT = 640
NBT = (N_NODES + BT - 1) // BT


def _graph_body(net_ref, w1_ref, b1_ref, w2_ref, b2_ref, out_ref, pad_ref,
                acc_ref):
  k = pl.program_id(0)

  @pl.when(k == 0)
  def _():
    acc_ref[...] = jnp.zeros_like(acc_ref)

  x = net_ref[...]
  pad_ref[:, :D] = jnp.swapaxes(x, 0, 1)
  pad_ref[:, D:] = jnp.zeros((BT, DP - D), jnp.float32)
  col = k * BT + jax.lax.broadcasted_iota(jnp.int32, (D, BT), 1)
  xm = jnp.where(col < N_NODES, x, 0.0)
  acc_ref[...] += jnp.sum(xm, axis=1, keepdims=True)

  @pl.when(k == NBT - 1)
  def _():
    hg = acc_ref[...] * jnp.float32(1.0 / N_NODES)
    hgb = hg.astype(jnp.bfloat16).astype(jnp.float32)
    w1b = w1_ref[...].astype(jnp.bfloat16).astype(jnp.float32)
    h1 = jnp.maximum(
        lax.dot_general(w1b, hgb, (((0,), (0,)), ((), ())),
                        preferred_element_type=jnp.float32,
                        precision=lax.Precision.HIGHEST)
        + b1_ref[...], 0.0)
    out_ref[...] = (
        lax.dot_general(w2_ref[...], h1, (((0,), (0,)), ((), ())),
                        preferred_element_type=jnp.float32,
                        precision=lax.Precision.HIGHEST)
        + b2_ref[...])


def _graph_tc(node_emb_t, w1, b1, w2, b2):
  out, padded = pl.pallas_call(
      _graph_body,
      grid=(NBT,),
      in_specs=[
          pl.BlockSpec((D, BT), lambda k: (0, k)),
          pl.BlockSpec((D, D // 2), lambda k: (0, 0)),
          pl.BlockSpec((D // 2, 1), lambda k: (0, 0)),
          pl.BlockSpec((D // 2, 1), lambda k: (0, 0)),
          pl.BlockSpec((1, 1), lambda k: (0, 0)),
      ],
      out_specs=[
          pl.BlockSpec((1, 1), lambda k: (0, 0)),
          pl.BlockSpec((BT, DP), lambda k: (k, 0)),
      ],
      out_shape=[
          jax.ShapeDtypeStruct((1, 1), jnp.float32),
          jax.ShapeDtypeStruct((N_NODES, DP), jnp.float32),
      ],
      scratch_shapes=[pltpu.VMEM((D, 1), jnp.float32)],
  )(node_emb_t, w1, b1.reshape(-1, 1), w2, b2.reshape(1, 1))
  return out.reshape(1), padded


def kernel(node_emb, rel_mat, W1, b1, W2, b2, generate_neighbor_emb,
           pos_edges, neg1_edges, neg2_edges):
  src_idx = jnp.concatenate(
      [pos_edges[:, 0], neg1_edges[:, 0], neg2_edges[:, 0]], axis=1)
  dst_idx = jnp.concatenate([pos_edges[:, 1], neg1_edges[:, 1]], axis=1)

  rel_mat_p = jnp.pad(rel_mat, ((0, 0), (0, DP - D), (0, DP - D)))
  graph_embd, node_emb_p = _graph_tc(node_emb.T, W1, b1, W2, b2)

  chunk_scores = []
  for c in range(N_CHUNKS):
    idx_c = jnp.concatenate([
        src_idx[c * R_CHUNK:(c + 1) * R_CHUNK].reshape(-1),
        dst_idx[c * R_CHUNK:(c + 1) * R_CHUNK].reshape(-1),
    ])
    idx_c = jnp.pad(idx_c, (0, N_PAD_C - ROWS_C)).reshape(NW, ITERS, CHUNK)
    gathered_c = _sc_gather(node_emb_p, idx_c)
    chunk_scores.append(
        _score_tc(gathered_c, generate_neighbor_emb, rel_mat_p, c))

  scores = jnp.concatenate(chunk_scores, axis=0)
  pos_score = scores[:, :E].reshape(-1)
  neg_score1 = scores[:, E:2 * E].reshape(-1)
  neg_score2 = scores[:, 2 * E:].reshape(-1)
  return (pos_score, neg_score1, neg_score2, graph_embd)
